# Initial kernel scaffold; baseline (speedup 1.0000x reference)
import jax
import jax.numpy as jnp
from jax.experimental import pallas as pl


def kernel(x, mask, W, b):
    raise NotImplementedError("write your pallas kernel here")


# speedup vs baseline: 2.0455x; 2.0455x over previous
import jax
import jax.numpy as jnp
from jax.experimental import pallas as pl


def _conv1x1_block(x_ref, w_ref, b_ref, o_ref):
    o_ref[0] = (
        jnp.dot(w_ref[...], x_ref[0], preferred_element_type=jnp.float32)
        + b_ref[...]
    )


def kernel(x, mask, W, b):
    N, C, H, Wsp = x.shape
    P = H * Wsp
    x2 = x.reshape(N, C, P)
    W2 = W.reshape(C, C)
    b2 = b.reshape(C, 1)

    T = 8192
    grid = (N, P // T)

    out = pl.pallas_call(
        _conv1x1_block,
        grid=grid,
        in_specs=[
            pl.BlockSpec((1, C, T), lambda n, j: (n, 0, j)),
            pl.BlockSpec((C, C), lambda n, j: (0, 0)),
            pl.BlockSpec((C, 1), lambda n, j: (0, 0)),
        ],
        out_specs=pl.BlockSpec((1, C, T), lambda n, j: (n, 0, j)),
        out_shape=jax.ShapeDtypeStruct((N, C, P), jnp.float32),
    )(x2, W2, b2)
    return out.reshape(N, C, H, Wsp)


# speedup vs baseline: 2.0715x; 1.0128x over previous
import jax
import jax.numpy as jnp
from jax.experimental import pallas as pl


def _conv1x1_block(x_ref, w_ref, b_ref, o_ref):
    o_ref[0] = (
        jnp.dot(w_ref[...], x_ref[0], preferred_element_type=jnp.float32)
        + b_ref[...]
    )


def kernel(x, mask, W, b):
    N, C, H, Wsp = x.shape
    P = H * Wsp
    x2 = x.reshape(N, C, P)
    W2 = W.reshape(C, C)
    b2 = b.reshape(C, 1)

    T = 16384
    grid = (N, P // T)

    out = pl.pallas_call(
        _conv1x1_block,
        grid=grid,
        in_specs=[
            pl.BlockSpec((1, C, T), lambda n, j: (n, 0, j)),
            pl.BlockSpec((C, C), lambda n, j: (0, 0)),
            pl.BlockSpec((C, 1), lambda n, j: (0, 0)),
        ],
        out_specs=pl.BlockSpec((1, C, T), lambda n, j: (n, 0, j)),
        out_shape=jax.ShapeDtypeStruct((N, C, P), jnp.float32),
    )(x2, W2, b2)
    return out.reshape(N, C, H, Wsp)
